# baseline (speedup 1.0000x reference)
import functools

import jax
import jax.numpy as jnp
from jax import lax
from jax.experimental import pallas as pl
from jax.experimental.pallas import tpu as pltpu
from jax.experimental.pallas import tpu_sc as plsc

_NUM_CLASSES = 150
_SMOOTH = 1e-05
_NHI = 16
_NLO = 16
_BINS = _NHI * _NLO

_H_SC = 48
_NC = 2
_NS = 16
_NW = _NC * _NS
_L = 16
_W = 384
_G = 8


def _hist2d(v_flat, n):
    hi = v_flat >> 4
    lo = v_flat & 15
    hi_iota = lax.broadcasted_iota(jnp.int32, (_NHI, n), 0)
    lo_iota = lax.broadcasted_iota(jnp.int32, (_NLO, n), 0)
    one = jnp.float32(1.0)
    zero = jnp.float32(0.0)
    hi_f = jnp.where(hi == hi_iota, one, zero)
    lo_f = jnp.where(lo == lo_iota, one, zero)
    return lax.dot_general(
        hi_f, lo_f, (((1,), (1,)), ((), ())),
        preferred_element_type=jnp.float32,
    )


def _hist3(pred_flat, t_flat, n):
    eq = t_flat == pred_flat
    t_masked = jnp.where(eq, t_flat, _NUM_CLASSES)
    return _hist2d(t_masked, n), _hist2d(pred_flat, n), _hist2d(t_flat, n)


def _tc_kernel(x_ref, t_ref, out_ref, acc_ref):
    b = pl.program_id(0)
    i = pl.program_id(1)
    nb = pl.num_programs(0)
    ni = pl.num_programs(1)

    @pl.when((b == 0) & (i == 0))
    def _init():
        acc_ref[...] = jnp.zeros_like(acc_ref)

    x = x_ref[0]
    t = t_ref[0]
    c, r, w = x.shape
    n = r * w

    runmax = x[0]
    runidx = jnp.zeros((r, w), jnp.int32)
    for ci in range(1, c):
        xi = x[ci]
        gt = xi > runmax
        runmax = jnp.maximum(runmax, xi)
        runidx = jnp.where(gt, ci, runidx)

    h_i, h_p, h_t = _hist3(runidx.reshape(1, n), t.reshape(1, n), n)
    acc_ref[0] += h_i
    acc_ref[1] += h_p
    acc_ref[2] += h_t

    @pl.when((b == nb - 1) & (i == ni - 1))
    def _fin():
        out_ref[...] = acc_ref[...]


def _tc_counts(x, t):
    B, C, H, W = x.shape
    R = 48
    nblk = H // R
    return pl.pallas_call(
        _tc_kernel,
        grid=(B, nblk),
        in_specs=[
            pl.BlockSpec((1, C, R, W), lambda b, i: (b, 0, i, 0)),
            pl.BlockSpec((1, R, W), lambda b, i: (b, i, 0)),
        ],
        out_specs=pl.BlockSpec((3, _NHI, _NLO), lambda b, i: (0, 0, 0)),
        out_shape=jax.ShapeDtypeStruct((3, _NHI, _NLO), jnp.float32),
        scratch_shapes=[pltpu.VMEM((3, _NHI, _NLO), jnp.float32)],
    )(x, t)


def _sc_body(x_hbm, out_hbm, xrow_v, prow_v):
    cid = lax.axis_index("c")
    sid = lax.axis_index("s")
    wid = sid * _NC + cid
    c = _NUM_CLASSES
    nru = 2 * _H_SC // _NW

    for u in range(nru):
        ru = wid * nru + u
        b = ru // _H_SC
        r = ru - b * _H_SC
        pltpu.sync_copy(x_hbm.at[b, :, r, :], xrow_v)

        for g0 in range(0, _W // _L, _G):
            cols = [(g0 + k) * _L for k in range(_G)]

            def _amax(ci, carry):
                out = []
                for k in range(_G):
                    rm, ri = carry[2 * k], carry[2 * k + 1]
                    xi = xrow_v[ci, pl.ds(cols[k], _L)]
                    gt = xi > rm
                    out.append(jnp.where(gt, xi, rm))
                    out.append(jnp.where(gt, ci, ri))
                return tuple(out)

            init = []
            for k in range(_G):
                init.append(xrow_v[0, pl.ds(cols[k], _L)])
                init.append(jnp.zeros((_L,), jnp.int32))
            res = lax.fori_loop(1, c, _amax, tuple(init))

            for k in range(_G):
                prow_v[pl.ds(cols[k], _L)] = res[2 * k + 1]

        pltpu.sync_copy(prow_v, out_hbm.at[b, r, :])


def _sc_pred(x_sc):
    B = x_sc.shape[0]
    mesh = plsc.VectorSubcoreMesh(core_axis_name="c", subcore_axis_name="s")
    k = functools.partial(
        pl.kernel,
        mesh=mesh,
        out_type=jax.ShapeDtypeStruct((B, _H_SC, _W), jnp.int32),
        scratch_types=[
            pltpu.VMEM((_NUM_CLASSES, _W), jnp.float32),
            pltpu.VMEM((_W,), jnp.int32),
        ],
    )(_sc_body)
    return k(x_sc)


def _combine_kernel(tc_ref, p_ref, t_ref, out_ref):
    tc = tc_ref[...]
    n = p_ref.shape[1]
    h_i, h_p, h_t = _hist3(p_ref[...], t_ref[...], n)
    inter = tc[0] + h_i
    cnt_p = tc[1] + h_p
    cnt_t = tc[2] + h_t
    out_ref[...] = inter / (cnt_p + cnt_t - inter + _SMOOTH)


def _combine(tc_acc, sc_pred, t_sc):
    n = sc_pred.size
    iou2d = pl.pallas_call(
        _combine_kernel,
        out_shape=jax.ShapeDtypeStruct((_NHI, _NLO), jnp.float32),
    )(tc_acc, sc_pred.reshape(1, n), t_sc.reshape(1, n))
    return iou2d.reshape(_BINS)[:_NUM_CLASSES]


def kernel(pred_logits, target):
    B, C, H, W = pred_logits.shape
    h_tc = H - _H_SC
    sc_pred = _sc_pred(pred_logits[:, :, h_tc:, :])
    tc_acc = _tc_counts(pred_logits[:, :, :h_tc, :], target[:, :h_tc, :])
    return _combine(tc_acc, sc_pred, target[:, h_tc:, :])


# speedup vs baseline: 2.2913x; 2.2913x over previous
import jax
import jax.numpy as jnp
from jax.experimental import pallas as pl
from jax.experimental.pallas import tpu as pltpu

_NUM_CLASSES = 150
_SMOOTH = 1e-05
_NHI = 16
_NLO = 16
_CK = 6


def _hist2d(v_flat, n):
    hi = v_flat >> 4
    lo = v_flat & 15
    hi_iota = jax.lax.broadcasted_iota(jnp.int32, (_NHI, n), 0)
    lo_iota = jax.lax.broadcasted_iota(jnp.int32, (_NLO, n), 0)
    one = jnp.float32(1.0)
    zero = jnp.float32(0.0)
    hi_f = jnp.where(hi == hi_iota, one, zero)
    lo_f = jnp.where(lo == lo_iota, one, zero)
    return jax.lax.dot_general(
        hi_f, lo_f, (((1,), (1,)), ((), ())),
        preferred_element_type=jnp.float32,
    )


def _iou_kernel(x_ref, t_ref, out_ref, rmax_ref, ridx_ref, acc_ref):
    b = pl.program_id(0)
    j = pl.program_id(1)
    nb = pl.num_programs(0)
    nj = pl.num_programs(1)

    @pl.when((b == 0) & (j == 0))
    def _init_acc():
        acc_ref[...] = jnp.zeros_like(acc_ref)

    @pl.when(j == 0)
    def _init_state():
        rmax_ref[...] = jnp.full_like(rmax_ref, -jnp.inf)
        ridx_ref[...] = jnp.zeros_like(ridx_ref)

    x = x_ref[0]
    ck, h, w = x.shape
    n = h * w

    rm = rmax_ref[...]
    ri = ridx_ref[...]
    base = j * ck
    for ci in range(ck):
        xi = x[ci]
        gt = xi > rm
        rm = jnp.maximum(rm, xi)
        ri = jnp.where(gt, base + ci, ri)
    rmax_ref[...] = rm
    ridx_ref[...] = ri

    @pl.when(j == nj - 1)
    def _hist():
        t = t_ref[0]
        t_flat = t.reshape(1, n)
        p_flat = ri.reshape(1, n)
        eq = t_flat == p_flat
        t_masked = jnp.where(eq, t_flat, _NUM_CLASSES)
        acc_ref[0] += _hist2d(t_masked, n)
        acc_ref[1] += _hist2d(p_flat, n)
        acc_ref[2] += _hist2d(t_flat, n)

    @pl.when((b == nb - 1) & (j == nj - 1))
    def _fin():
        inter = acc_ref[0].reshape(1, _NHI * _NLO)
        cnt_p = acc_ref[1].reshape(1, _NHI * _NLO)
        cnt_t = acc_ref[2].reshape(1, _NHI * _NLO)
        out_ref[...] = (inter / (cnt_p + cnt_t - inter + _SMOOTH))[0, :_NUM_CLASSES]


def kernel(pred_logits, target):
    B, C, H, W = pred_logits.shape
    nj = C // _CK
    out = pl.pallas_call(
        _iou_kernel,
        grid=(B, nj),
        in_specs=[
            pl.BlockSpec((1, _CK, H, W), lambda b, j: (b, j, 0, 0)),
            pl.BlockSpec((1, H, W), lambda b, j: (b, 0, 0)),
        ],
        out_specs=pl.BlockSpec((C,), lambda b, j: (0,)),
        out_shape=jax.ShapeDtypeStruct((C,), jnp.float32),
        scratch_shapes=[
            pltpu.VMEM((H, W), jnp.float32),
            pltpu.VMEM((H, W), jnp.int32),
            pltpu.VMEM((3, _NHI, _NLO), jnp.float32),
        ],
    )(pred_logits, target)
    return out


# speedup vs baseline: 3.4613x; 1.5106x over previous
import jax
import jax.numpy as jnp
from jax.experimental import pallas as pl
from jax.experimental.pallas import tpu as pltpu

_NUM_CLASSES = 150
_SMOOTH = 1e-05
_NHI = 16
_NLO = 16


def _hist2d(v_flat, n):
    hi = v_flat >> 4
    lo = v_flat & 15
    hi_iota = jax.lax.broadcasted_iota(jnp.int32, (_NHI, n), 0)
    lo_iota = jax.lax.broadcasted_iota(jnp.int32, (_NLO, n), 0)
    one = jnp.float32(1.0)
    zero = jnp.float32(0.0)
    hi_f = jnp.where(hi == hi_iota, one, zero)
    lo_f = jnp.where(lo == lo_iota, one, zero)
    return jax.lax.dot_general(
        hi_f, lo_f, (((1,), (1,)), ((), ())),
        preferred_element_type=jnp.float32,
    )


def _iou_kernel(x_ref, t_ref, out_ref, acc_ref):
    b = pl.program_id(0)
    i = pl.program_id(1)
    nb = pl.num_programs(0)
    ni = pl.num_programs(1)

    @pl.when((b == 0) & (i == 0))
    def _init():
        acc_ref[...] = jnp.zeros_like(acc_ref)

    x = x_ref[0]
    t = t_ref[0]
    c, r, w = x.shape
    n = r * w

    runmax = x[0]
    runidx = jnp.zeros((r, w), jnp.int32)
    for ci in range(1, c):
        xi = x[ci]
        gt = xi > runmax
        runmax = jnp.maximum(runmax, xi)
        runidx = jnp.where(gt, ci, runidx)
    pred = runidx

    t_flat = t.reshape(1, n)
    p_flat = pred.reshape(1, n)
    eq = t_flat == p_flat
    t_masked = jnp.where(eq, t_flat, c)

    acc_ref[0] += _hist2d(t_masked, n)
    acc_ref[1] += _hist2d(p_flat, n)
    acc_ref[2] += _hist2d(t_flat, n)

    @pl.when((b == nb - 1) & (i == ni - 1))
    def _fin():
        inter = acc_ref[0].reshape(1, _NHI * _NLO)
        cnt_p = acc_ref[1].reshape(1, _NHI * _NLO)
        cnt_t = acc_ref[2].reshape(1, _NHI * _NLO)
        out_ref[...] = (inter / (cnt_p + cnt_t - inter + _SMOOTH))[0, :_NUM_CLASSES]


def kernel(pred_logits, target):
    B, C, H, W = pred_logits.shape
    R = 64
    nblk = H // R
    out = pl.pallas_call(
        _iou_kernel,
        grid=(B, nblk),
        in_specs=[
            pl.BlockSpec((1, C, R, W), lambda b, i: (b, 0, i, 0)),
            pl.BlockSpec((1, R, W), lambda b, i: (b, i, 0)),
        ],
        out_specs=pl.BlockSpec((C,), lambda b, i: (0,)),
        out_shape=jax.ShapeDtypeStruct((C,), jnp.float32),
        scratch_shapes=[pltpu.VMEM((3, _NHI, _NLO), jnp.float32)],
    )(pred_logits, target)
    return out
